# baseline (speedup 1.0000x reference)
import jax
import jax.numpy as jnp
from jax.experimental import pallas as pl
from jax.experimental.pallas import tpu as pltpu

_EPS = 1e-6


def _ln_kernel(gamma_ref, beta_ref, x_ref, o_ref):
    x = x_ref[...].astype(jnp.float32)
    h = x.shape[-1]
    s1 = jnp.sum(x, axis=-1, keepdims=True)
    s2 = jnp.sum(x * x, axis=-1, keepdims=True)
    mean = s1 * (1.0 / h)
    ssq = s2 - s1 * mean
    std = jnp.sqrt(ssq * (1.0 / max(h - 1, 1)))
    scale = gamma_ref[0, 0] * pl.reciprocal(std + _EPS, approx=True)
    o_ref[...] = ((x - mean) * scale + beta_ref[0, 0]).astype(o_ref.dtype)


def _layer_norm(x, gamma, beta, *, tile_rows=2048):
    orig_shape = x.shape
    H = orig_shape[-1]
    xf = x.reshape(-1, H)
    R = xf.shape[0]
    dtype = x.dtype

    g = jnp.asarray(gamma, jnp.float32).reshape(1, 1)
    b = jnp.asarray(beta, jnp.float32).reshape(1, 1)

    tile_rows = min(tile_rows, max(8, -(-R // 8) * 8))
    num_tiles = pl.cdiv(R, tile_rows)
    padded_rows = num_tiles * tile_rows
    if padded_rows != R:
        xf = jnp.pad(xf, ((0, padded_rows - R), (0, 0)))

    smem = pl.BlockSpec(memory_space=pltpu.MemorySpace.SMEM)
    inner = num_tiles // 2 if num_tiles % 2 == 0 else num_tiles
    ncore = num_tiles // inner
    out = pl.pallas_call(
        _ln_kernel,
        out_shape=jax.ShapeDtypeStruct((padded_rows, H), dtype),
        grid=(ncore, inner),
        in_specs=[smem, smem,
                  pl.BlockSpec((tile_rows, H), lambda c, i: (c * inner + i, 0))],
        out_specs=pl.BlockSpec((tile_rows, H), lambda c, i: (c * inner + i, 0)),
        compiler_params=pltpu.CompilerParams(
            dimension_semantics=("parallel", "arbitrary"),
            vmem_limit_bytes=64 << 20,
        ),
    )(g, b, xf)

    return out[:R].reshape(orig_shape)


def kernel(x, gamma, beta):
    return _layer_norm(x, gamma, beta)


# speedup vs baseline: 1.1040x; 1.1040x over previous
import jax
import jax.numpy as jnp
from jax.experimental import pallas as pl
from jax.experimental.pallas import tpu as pltpu

_EPS = 1e-6
_NBUF = 4
_LOOKAHEAD = 3


def _ln_chunk(x, h, gamma, beta):
    s1 = jnp.sum(x, axis=-1, keepdims=True)
    s2 = jnp.sum(x * x, axis=-1, keepdims=True)
    mean = s1 * (1.0 / h)
    ssq = s2 - s1 * mean
    std = jnp.sqrt(ssq * (1.0 / max(h - 1, 1)))
    scale = gamma * pl.reciprocal(std + _EPS, approx=True)
    return (x - mean) * scale + beta


def _ln_manual_kernel(rows_per_core, chunk_rows, gamma_ref, beta_ref,
                      x_hbm, o_hbm, in_buf, out_buf, load_sem, store_sem):
    core = pl.program_id(0)
    base = core * rows_per_core
    nch = rows_per_core // chunk_rows
    h = in_buf.shape[-1]

    def load(i):
        slot = i % _NBUF
        pltpu.make_async_copy(
            x_hbm.at[pl.ds(base + i * chunk_rows, chunk_rows), :],
            in_buf.at[slot], load_sem.at[slot]).start()

    def store(i):
        slot = i % _NBUF
        pltpu.make_async_copy(
            out_buf.at[slot],
            o_hbm.at[pl.ds(base + i * chunk_rows, chunk_rows), :],
            store_sem.at[slot]).start()

    for i in range(min(_LOOKAHEAD, nch)):
        load(i)

    gamma = gamma_ref[0, 0]
    beta = beta_ref[0, 0]
    for i in range(nch):
        slot = i % _NBUF
        if i + _LOOKAHEAD < nch:
            load(i + _LOOKAHEAD)
        pltpu.make_async_copy(
            x_hbm.at[pl.ds(base + i * chunk_rows, chunk_rows), :],
            in_buf.at[slot], load_sem.at[slot]).wait()
        if i >= _NBUF:
            pltpu.make_async_copy(
                out_buf.at[slot],
                o_hbm.at[pl.ds(base + (i - _NBUF) * chunk_rows, chunk_rows), :],
                store_sem.at[slot]).wait()
        out_buf[slot] = _ln_chunk(in_buf[slot], h, gamma, beta)
        store(i)

    for i in range(max(0, nch - _NBUF), nch):
        slot = i % _NBUF
        pltpu.make_async_copy(
            out_buf.at[slot],
            o_hbm.at[pl.ds(base + i * chunk_rows, chunk_rows), :],
            store_sem.at[slot]).wait()


def _layer_norm(x, gamma, beta, *, chunk_rows=512):
    orig_shape = x.shape
    H = orig_shape[-1]
    xf = x.reshape(-1, H)
    R = xf.shape[0]
    dtype = x.dtype

    g = jnp.asarray(gamma, jnp.float32).reshape(1, 1)
    b = jnp.asarray(beta, jnp.float32).reshape(1, 1)

    rows_per_core = -(-R // 2)
    chunk_rows = min(chunk_rows, max(8, -(-rows_per_core // 8) * 8))
    nch = -(-rows_per_core // chunk_rows)
    rows_per_core = nch * chunk_rows
    padded_rows = 2 * rows_per_core
    if padded_rows != R:
        xf = jnp.pad(xf, ((0, padded_rows - R), (0, 0)))

    import functools
    body = functools.partial(_ln_manual_kernel, rows_per_core, chunk_rows)
    smem = pl.BlockSpec(memory_space=pltpu.MemorySpace.SMEM)
    hbm = pl.BlockSpec(memory_space=pl.ANY)
    out = pl.pallas_call(
        body,
        out_shape=jax.ShapeDtypeStruct((padded_rows, H), dtype),
        grid=(2,),
        in_specs=[smem, smem, hbm],
        out_specs=hbm,
        scratch_shapes=[
            pltpu.VMEM((_NBUF, chunk_rows, H), jnp.float32),
            pltpu.VMEM((_NBUF, chunk_rows, H), jnp.float32),
            pltpu.SemaphoreType.DMA((_NBUF,)),
            pltpu.SemaphoreType.DMA((_NBUF,)),
        ],
        compiler_params=pltpu.CompilerParams(
            dimension_semantics=("parallel",),
            vmem_limit_bytes=64 << 20,
        ),
    )(g, b, xf)

    return out[:R].reshape(orig_shape)


def kernel(x, gamma, beta):
    return _layer_norm(x, gamma, beta)


# speedup vs baseline: 1.1441x; 1.0363x over previous
import jax
import jax.numpy as jnp
from jax.experimental import pallas as pl
from jax.experimental.pallas import tpu as pltpu

_EPS = 1e-6
_NBUF = 4
_LOOKAHEAD = 3


def _ln_chunk(x, h, gamma, beta):
    s1 = jnp.sum(x, axis=-1, keepdims=True)
    s2 = jnp.sum(x * x, axis=-1, keepdims=True)
    mean = s1 * (1.0 / h)
    ssq = s2 - s1 * mean
    std = jnp.sqrt(ssq * (1.0 / max(h - 1, 1)))
    scale = gamma * pl.reciprocal(std + _EPS, approx=True)
    return (x - mean) * scale + beta


def _ln_manual_kernel(rows_per_core, chunk_rows, gamma_ref, beta_ref,
                      x_hbm, o_hbm, in_buf, out_buf, load_sem, store_sem):
    core = pl.program_id(0)
    base = core * rows_per_core
    nch = rows_per_core // chunk_rows
    h = in_buf.shape[-1]

    def load(i):
        slot = i % _NBUF
        pltpu.make_async_copy(
            x_hbm.at[pl.ds(base + i * chunk_rows, chunk_rows), :],
            in_buf.at[slot], load_sem.at[slot]).start()

    def store(i):
        slot = i % _NBUF
        pltpu.make_async_copy(
            out_buf.at[slot],
            o_hbm.at[pl.ds(base + i * chunk_rows, chunk_rows), :],
            store_sem.at[slot]).start()

    for i in range(min(_LOOKAHEAD, nch)):
        load(i)

    gamma = gamma_ref[0, 0]
    beta = beta_ref[0, 0]
    for i in range(nch):
        slot = i % _NBUF
        if i + _LOOKAHEAD < nch:
            load(i + _LOOKAHEAD)
        pltpu.make_async_copy(
            x_hbm.at[pl.ds(base + i * chunk_rows, chunk_rows), :],
            in_buf.at[slot], load_sem.at[slot]).wait()
        if i >= _NBUF:
            pltpu.make_async_copy(
                out_buf.at[slot],
                o_hbm.at[pl.ds(base + (i - _NBUF) * chunk_rows, chunk_rows), :],
                store_sem.at[slot]).wait()
        out_buf[slot] = _ln_chunk(in_buf[slot], h, gamma, beta)
        store(i)

    for i in range(max(0, nch - _NBUF), nch):
        slot = i % _NBUF
        pltpu.make_async_copy(
            out_buf.at[slot],
            o_hbm.at[pl.ds(base + i * chunk_rows, chunk_rows), :],
            store_sem.at[slot]).wait()


def _layer_norm(x, gamma, beta, *, chunk_rows=1024):
    orig_shape = x.shape
    H = orig_shape[-1]
    xf = x.reshape(-1, H)
    R = xf.shape[0]
    dtype = x.dtype

    g = jnp.asarray(gamma, jnp.float32).reshape(1, 1)
    b = jnp.asarray(beta, jnp.float32).reshape(1, 1)

    rows_per_core = -(-R // 2)
    chunk_rows = min(chunk_rows, max(8, -(-rows_per_core // 8) * 8))
    nch = -(-rows_per_core // chunk_rows)
    rows_per_core = nch * chunk_rows
    padded_rows = 2 * rows_per_core
    if padded_rows != R:
        xf = jnp.pad(xf, ((0, padded_rows - R), (0, 0)))

    import functools
    body = functools.partial(_ln_manual_kernel, rows_per_core, chunk_rows)
    smem = pl.BlockSpec(memory_space=pltpu.MemorySpace.SMEM)
    hbm = pl.BlockSpec(memory_space=pl.ANY)
    out = pl.pallas_call(
        body,
        out_shape=jax.ShapeDtypeStruct((padded_rows, H), dtype),
        grid=(2,),
        in_specs=[smem, smem, hbm],
        out_specs=hbm,
        scratch_shapes=[
            pltpu.VMEM((_NBUF, chunk_rows, H), jnp.float32),
            pltpu.VMEM((_NBUF, chunk_rows, H), jnp.float32),
            pltpu.SemaphoreType.DMA((_NBUF,)),
            pltpu.SemaphoreType.DMA((_NBUF,)),
        ],
        compiler_params=pltpu.CompilerParams(
            dimension_semantics=("parallel",),
            vmem_limit_bytes=64 << 20,
        ),
    )(g, b, xf)

    return out[:R].reshape(orig_shape)


def kernel(x, gamma, beta):
    return _layer_norm(x, gamma, beta)


# speedup vs baseline: 1.1491x; 1.0044x over previous
import jax
import jax.numpy as jnp
from jax.experimental import pallas as pl
from jax.experimental.pallas import tpu as pltpu

_EPS = 1e-6
_NBUF = 6
_LOOKAHEAD = 5


def _ln_chunk(x, h, gamma, beta):
    s1 = jnp.sum(x, axis=-1, keepdims=True)
    s2 = jnp.sum(x * x, axis=-1, keepdims=True)
    mean = s1 * (1.0 / h)
    ssq = s2 - s1 * mean
    std = jnp.sqrt(ssq * (1.0 / max(h - 1, 1)))
    scale = gamma * pl.reciprocal(std + _EPS, approx=True)
    return (x - mean) * scale + beta


def _ln_manual_kernel(rows_per_core, chunk_rows, gamma_ref, beta_ref,
                      x_hbm, o_hbm, in_buf, out_buf, load_sem, store_sem):
    core = pl.program_id(0)
    base = core * rows_per_core
    nch = rows_per_core // chunk_rows
    h = in_buf.shape[-1]

    def load(i):
        slot = i % _NBUF
        pltpu.make_async_copy(
            x_hbm.at[pl.ds(base + i * chunk_rows, chunk_rows), :],
            in_buf.at[slot], load_sem.at[slot]).start()

    def store(i):
        slot = i % _NBUF
        pltpu.make_async_copy(
            out_buf.at[slot],
            o_hbm.at[pl.ds(base + i * chunk_rows, chunk_rows), :],
            store_sem.at[slot]).start()

    for i in range(min(_LOOKAHEAD, nch)):
        load(i)

    gamma = gamma_ref[0, 0]
    beta = beta_ref[0, 0]
    for i in range(nch):
        slot = i % _NBUF
        if i + _LOOKAHEAD < nch:
            load(i + _LOOKAHEAD)
        pltpu.make_async_copy(
            x_hbm.at[pl.ds(base + i * chunk_rows, chunk_rows), :],
            in_buf.at[slot], load_sem.at[slot]).wait()
        if i >= _NBUF:
            pltpu.make_async_copy(
                out_buf.at[slot],
                o_hbm.at[pl.ds(base + (i - _NBUF) * chunk_rows, chunk_rows), :],
                store_sem.at[slot]).wait()
        out_buf[slot] = _ln_chunk(in_buf[slot], h, gamma, beta)
        store(i)

    for i in range(max(0, nch - _NBUF), nch):
        slot = i % _NBUF
        pltpu.make_async_copy(
            out_buf.at[slot],
            o_hbm.at[pl.ds(base + i * chunk_rows, chunk_rows), :],
            store_sem.at[slot]).wait()


def _layer_norm(x, gamma, beta, *, chunk_rows=1024):
    orig_shape = x.shape
    H = orig_shape[-1]
    xf = x.reshape(-1, H)
    R = xf.shape[0]
    dtype = x.dtype

    g = jnp.asarray(gamma, jnp.float32).reshape(1, 1)
    b = jnp.asarray(beta, jnp.float32).reshape(1, 1)

    rows_per_core = -(-R // 2)
    chunk_rows = min(chunk_rows, max(8, -(-rows_per_core // 8) * 8))
    nch = -(-rows_per_core // chunk_rows)
    rows_per_core = nch * chunk_rows
    padded_rows = 2 * rows_per_core
    if padded_rows != R:
        xf = jnp.pad(xf, ((0, padded_rows - R), (0, 0)))

    import functools
    body = functools.partial(_ln_manual_kernel, rows_per_core, chunk_rows)
    smem = pl.BlockSpec(memory_space=pltpu.MemorySpace.SMEM)
    hbm = pl.BlockSpec(memory_space=pl.ANY)
    out = pl.pallas_call(
        body,
        out_shape=jax.ShapeDtypeStruct((padded_rows, H), dtype),
        grid=(2,),
        in_specs=[smem, smem, hbm],
        out_specs=hbm,
        scratch_shapes=[
            pltpu.VMEM((_NBUF, chunk_rows, H), jnp.float32),
            pltpu.VMEM((_NBUF, chunk_rows, H), jnp.float32),
            pltpu.SemaphoreType.DMA((_NBUF,)),
            pltpu.SemaphoreType.DMA((_NBUF,)),
        ],
        compiler_params=pltpu.CompilerParams(
            dimension_semantics=("parallel",),
            vmem_limit_bytes=64 << 20,
        ),
    )(g, b, xf)

    return out[:R].reshape(orig_shape)


def kernel(x, gamma, beta):
    return _layer_norm(x, gamma, beta)
